# baseline (speedup 1.0000x reference)
import jax
import jax.numpy as jnp
from jax.experimental import pallas as pl
from jax.experimental.pallas import tpu as pltpu

K = 2048
N = 100000
D = 256
BLK = 5000


def _score_body(norm_ref, embs_ref, scores_ref):
    norm = norm_ref[0]
    s = jax.lax.dot(embs_ref[...], norm_ref)
    scores_ref[...] = s


def _matvec_body(scorer_ref, norm_ref, embs_ref, scores_ref):
    raw = jax.lax.dot(embs_ref[...], scorer_ref[...])
    scores_ref[...] = raw / norm_ref[0]


def _scores(node_embs, scorer, norm):
    grid = N // BLK
    return pl.pallas_call(
        _matvec_body,
        grid=(grid,),
        in_specs=[
            pl.BlockSpec((D, 1), lambda i: (0, 0)),
            pl.BlockSpec(memory_space=pltpu.SMEM),
            pl.BlockSpec((BLK, D), lambda i: (i, 0)),
        ],
        out_specs=pl.BlockSpec((BLK, 1), lambda i: (i, 0)),
        out_shape=jax.ShapeDtypeStruct((N, 1), jnp.float32),
    )(scorer, norm.reshape(1), node_embs)


def kernel(node_embs, scorer):
    norm = jnp.maximum(jnp.linalg.norm(scorer), 1e-6)
    scores = _scores(node_embs, scorer, norm)
    vals, topk_indices = jax.lax.top_k(scores.reshape(-1), K)
    out = node_embs[topk_indices] * jnp.tanh(scores[topk_indices].reshape(-1, 1))
    return out.T


# speedup vs baseline: 1.0252x; 1.0252x over previous
import functools

import jax
import jax.numpy as jnp
from jax import lax
from jax.experimental import pallas as pl
from jax.experimental.pallas import tpu as pltpu
from jax.experimental.pallas import tpu_sc as plsc

K = 2048
N = 100000
D = 256
BLK = 5000

NUM_TILES = 32
ROWS_PER_TILE = K // NUM_TILES


def _matvec_body(scorer_ref, norm_ref, embs_ref, scores_ref):
    raw = jax.lax.dot(embs_ref[...], scorer_ref[...])
    scores_ref[...] = raw / norm_ref[0]


def _scores(node_embs, scorer, norm):
    grid = N // BLK
    return pl.pallas_call(
        _matvec_body,
        grid=(grid,),
        in_specs=[
            pl.BlockSpec((D, 1), lambda i: (0, 0)),
            pl.BlockSpec(memory_space=pltpu.SMEM),
            pl.BlockSpec((BLK, D), lambda i: (i, 0)),
        ],
        out_specs=pl.BlockSpec((BLK, 1), lambda i: (i, 0)),
        out_shape=jax.ShapeDtypeStruct((N, 1), jnp.float32),
    )(scorer, norm.reshape(1), node_embs)


def _sc_gather_body(idx_hbm, embs_hbm, out_hbm, idx_v, rows_v, sem):
    wid = lax.axis_index("s") * 2 + lax.axis_index("c")
    base = wid * ROWS_PER_TILE
    pltpu.sync_copy(idx_hbm.at[pl.ds(base, ROWS_PER_TILE)], idx_v)
    pltpu.async_copy(embs_hbm.at[idx_v], rows_v, sem).wait()
    pltpu.sync_copy(rows_v, out_hbm.at[pl.ds(base, ROWS_PER_TILE)])


def _sc_gather(sorted_idx, node_embs):
    mesh = plsc.VectorSubcoreMesh(core_axis_name="c", subcore_axis_name="s")
    fn = functools.partial(
        pl.kernel,
        mesh=mesh,
        out_type=jax.ShapeDtypeStruct((K, D), jnp.float32),
        scratch_types=[
            pltpu.VMEM((ROWS_PER_TILE,), jnp.int32),
            pltpu.VMEM((ROWS_PER_TILE, D), jnp.float32),
            pltpu.SemaphoreType.DMA,
        ],
    )(_sc_gather_body)
    return fn(sorted_idx, node_embs)


def _scale_t_body(rows_ref, tanh_ref, out_ref):
    out_ref[...] = jnp.transpose(rows_ref[...] * tanh_ref[...], (1, 0))


def _scale_transpose(rows, sorted_tanh):
    return pl.pallas_call(
        _scale_t_body,
        out_shape=jax.ShapeDtypeStruct((D, K), jnp.float32),
    )(rows, sorted_tanh.reshape(K, 1))


def kernel(node_embs, scorer):
    norm = jnp.maximum(jnp.linalg.norm(scorer), 1e-6)
    scores = _scores(node_embs, scorer, norm)
    vals, topk_indices = jax.lax.top_k(scores.reshape(-1), K)
    sorted_tanh = jnp.tanh(vals)
    rows = _sc_gather(topk_indices.astype(jnp.int32), node_embs)
    return _scale_transpose(rows, sorted_tanh)


# speedup vs baseline: 1.0529x; 1.0270x over previous
import functools

import jax
import jax.numpy as jnp
from jax import lax
from jax.experimental import pallas as pl
from jax.experimental.pallas import tpu as pltpu
from jax.experimental.pallas import tpu_sc as plsc

K = 2048
N = 100000
D = 256
BLK = 5000

NP = 100352
TILES = 16
ROWS_T = NP // TILES // 128
PER_TILE = ROWS_T * 128
CMAX = 2560
DUMP = 8192

MININT = -2147483648
BISECT_B1 = False
BISECT_B2 = False


def _matvec_body(scorer_ref, norm_ref, embs_ref, scores_ref):
    raw = jax.lax.dot(embs_ref[...], scorer_ref[...])
    scores_ref[...] = raw / norm_ref[0]


def _scores(node_embs, scorer, norm):
    grid = N // BLK
    return pl.pallas_call(
        _matvec_body,
        grid=(grid,),
        in_specs=[
            pl.BlockSpec((D, 1), lambda i: (0, 0)),
            pl.BlockSpec(memory_space=pltpu.SMEM),
            pl.BlockSpec((BLK, D), lambda i: (i, 0)),
        ],
        out_specs=pl.BlockSpec((BLK, 1), lambda i: (i, 0)),
        out_shape=jax.ShapeDtypeStruct((N, 1), jnp.float32),
    )(scorer, norm.reshape(1), node_embs)


def _key_u32(f16):
    u = plsc.bitcast(f16, jnp.int32)
    k = jnp.where(u < 0, jnp.bitwise_not(u), jnp.bitwise_xor(u, jnp.int32(MININT)))
    return plsc.bitcast(k, jnp.uint32)


def _select_body(scores_hbm, key_out, idx_out, fk_v, keys_v, tgt_v, val_v,
                 hist_v, red_v, cnt_v, acnt_v, gh_v, zer_v, sem, sem2,
                 sh_hist, sh_cnt, sh_key, sh_idx):
    wid = lax.axis_index("s")
    base = wid * PER_TILE
    lanes = jnp.arange(16, dtype=jnp.int32)
    ones = jnp.ones((16,), jnp.int32)

    pltpu.sync_copy(scores_hbm.at[pl.ds(base, PER_TILE)], fk_v)

    def conv(j, _):
        ku = _key_u32(fk_v[pl.ds(j * 16, 16)])
        keys_v[j // 8, pl.ds((j % 8) * 16, 16)] = plsc.bitcast(ku, jnp.int32)
        return _

    lax.fori_loop(0, ROWS_T * 8, conv, None)

    hi = jnp.uint32(0)
    k_rem = jnp.int32(K)
    for lvl in range(4):
        s = 24 - 8 * lvl

        def zero(i, _):
            hist_v[pl.ds(i * 16, 16)] = jnp.zeros((16,), jnp.int32)
            return _

        lax.fori_loop(0, 256, zero, None)

        def scan(j, _, s=s, lvl=lvl, hi=hi):
            ku = plsc.bitcast(keys_v[j // 8, pl.ds((j % 8) * 16, 16)],
                              jnp.uint32)
            b = lax.convert_element_type(
                jnp.bitwise_and(jnp.right_shift(ku, jnp.uint32(s)),
                                jnp.uint32(0xFF)), jnp.int32)
            addr = lanes * 256 + b
            if lvl == 0:
                plsc.addupdate_scatter(hist_v, [addr], ones)
            else:
                act = jnp.right_shift(ku, jnp.uint32(s + 8)) == hi
                plsc.addupdate_scatter(hist_v, [addr], ones, mask=act)
            return _

        lax.fori_loop(0, ROWS_T * 8, scan, None)

        for c in range(16):
            acc = hist_v[pl.ds(c * 16, 16)]
            for l in range(1, 16):
                acc = acc + hist_v[pl.ds(l * 256 + c * 16, 16)]
            red_v[pl.ds(c * 16, 16)] = acc
        pltpu.sync_copy(red_v, sh_hist.at[lvl, wid])
        plsc.subcore_barrier()

        pltpu.sync_copy(sh_hist.at[lvl], gh_v)
        g = []
        for c in range(16):
            acc = gh_v[0, pl.ds(c * 16, 16)]
            for t in range(1, 16):
                acc = acc + gh_v[t, pl.ds(c * 16, 16)]
            g.append(acc)
        carry = jnp.int32(0)
        nge = jnp.int32(0)
        suf = [None] * 16
        for c in range(15, -1, -1):
            tot = jnp.sum(g[c])
            suf[c] = (tot - plsc.cumsum(g[c])) + g[c] + carry
            carry = carry + tot
            nge = nge + jnp.sum((suf[c] >= k_rem).astype(jnp.int32))
        b_star = nge - 1
        a_cnt = jnp.int32(0)
        for c in range(16):
            biota = lanes + 16 * c
            a_cnt = a_cnt + jnp.sum(jnp.where(biota > b_star, g[c], 0))
        k_rem = k_rem - a_cnt
        hi = jnp.bitwise_or(
            jnp.left_shift(hi, jnp.uint32(8)),
            lax.convert_element_type(b_star, jnp.uint32))

    v_thresh = hi
    if BISECT_B1:
        red_v[pl.ds(0, 16)] = plsc.bitcast(
            jnp.broadcast_to(v_thresh, (16,)), jnp.int32)
        sl = pl.ds(wid * (CMAX // TILES), CMAX // TILES)
        for i in range(CMAX // TILES // 16):
            zer_v[pl.ds(i * 16, 16)] = jnp.broadcast_to(
                plsc.bitcast(jnp.broadcast_to(v_thresh, (16,)),
                             jnp.int32)[0], (16,))
        pltpu.sync_copy(zer_v, key_out.at[sl])
        pltpu.sync_copy(zer_v, idx_out.at[sl])
        return

    def cnt_scan(j, cnt):
        ku = plsc.bitcast(keys_v[j // 8, pl.ds((j % 8) * 16, 16)], jnp.uint32)
        return cnt + jnp.sum((ku >= v_thresh).astype(jnp.int32))

    my_cnt = lax.fori_loop(0, ROWS_T * 8, cnt_scan, jnp.int32(0))
    cnt_v[...] = jnp.where(lanes == 0, my_cnt, 0)
    pltpu.sync_copy(cnt_v, sh_cnt.at[wid])
    for i in range(CMAX // TILES // 16):
        zer_v[pl.ds(i * 16, 16)] = jnp.zeros((16,), jnp.int32)
    pltpu.sync_copy(zer_v, sh_key.at[pl.ds(wid * (CMAX // TILES),
                                           CMAX // TILES)])
    pltpu.sync_copy(zer_v, sh_idx.at[pl.ds(wid * (CMAX // TILES),
                                           CMAX // TILES)])
    plsc.subcore_barrier()

    pltpu.sync_copy(sh_cnt, acnt_v)
    prefix = jnp.int32(0)
    for t in range(TILES):
        c_t = acnt_v[t, pl.ds(0, 16)][0]
        prefix = prefix + jnp.where(jnp.int32(t) < wid, c_t, 0)

    def tgt_scan(j, cnt):
        r, cc = j // 8, (j % 8) * 16
        ku = plsc.bitcast(keys_v[r, pl.ds(cc, 16)], jnp.uint32)
        m = ku >= v_thresh
        c16 = plsc.cumsum(m.astype(jnp.int32))
        pos = prefix + cnt + c16 - 1
        gidx = base + j * 16 + lanes
        dump = CMAX + jnp.bitwise_and(gidx, DUMP - 1)
        tgt = jnp.where(jnp.logical_and(m, pos < CMAX), pos, dump)
        tgt_v[r, pl.ds(cc, 16)] = tgt
        val_v[r, pl.ds(cc, 16)] = gidx
        return cnt + jnp.sum(m.astype(jnp.int32))

    lax.fori_loop(0, ROWS_T * 8, tgt_scan, jnp.int32(0))
    if not BISECT_B2:
        for r in range(ROWS_T):
            pltpu.sync_copy(keys_v.at[r], sh_key.at[tgt_v.at[r]])
            pltpu.sync_copy(val_v.at[r], sh_idx.at[tgt_v.at[r]])
    plsc.subcore_barrier()

    sl = pl.ds(wid * (CMAX // TILES), CMAX // TILES)
    pltpu.sync_copy(sh_key.at[sl], zer_v)
    pltpu.sync_copy(zer_v, key_out.at[sl])
    pltpu.sync_copy(sh_idx.at[sl], zer_v)
    pltpu.sync_copy(zer_v, idx_out.at[sl])


def _select(scores_flat):
    mesh = plsc.VectorSubcoreMesh(core_axis_name="c", subcore_axis_name="s",
                                  num_cores=1)
    fn = functools.partial(
        pl.kernel,
        mesh=mesh,
        compiler_params=pltpu.CompilerParams(needs_layout_passes=False),
        out_type=(jax.ShapeDtypeStruct((CMAX,), jnp.int32),
                  jax.ShapeDtypeStruct((CMAX,), jnp.int32)),
        scratch_types=[
            pltpu.VMEM((PER_TILE,), jnp.float32),
            pltpu.VMEM((ROWS_T, 128), jnp.int32),
            pltpu.VMEM((ROWS_T, 128), jnp.int32),
            pltpu.VMEM((ROWS_T, 128), jnp.int32),
            pltpu.VMEM((16 * 256,), jnp.int32),
            pltpu.VMEM((256,), jnp.int32),
            pltpu.VMEM((16,), jnp.int32),
            pltpu.VMEM((16, 16), jnp.int32),
            pltpu.VMEM((16, 256), jnp.int32),
            pltpu.VMEM((CMAX // TILES,), jnp.int32),
            pltpu.SemaphoreType.DMA,
            pltpu.SemaphoreType.DMA,
            pltpu.VMEM_SHARED((4, 16, 256), jnp.int32),
            pltpu.VMEM_SHARED((16, 16), jnp.int32),
            pltpu.VMEM_SHARED((CMAX + DUMP,), jnp.int32),
            pltpu.VMEM_SHARED((CMAX + DUMP,), jnp.int32),
        ],
    )(_select_body)
    return fn(scores_flat)


def _rank_body(kc_ref, ic_ref, kr_ref, ir_ref, rank_ref, tanh_ref):
    kc = jnp.bitwise_xor(kc_ref[...], jnp.int32(MININT))
    ic = ic_ref[...]
    rank = jnp.zeros((CMAX, 1), jnp.int32)
    CH = 512
    for ch in range(CMAX // CH):
        kr = jnp.bitwise_xor(kr_ref[:, ch * CH:(ch + 1) * CH], jnp.int32(MININT))
        ir = ir_ref[:, ch * CH:(ch + 1) * CH]
        gt = kr > kc
        tie = jnp.logical_and(kr == kc, ir < ic)
        cnt = jnp.logical_or(gt, tie).astype(jnp.int32)
        rank = rank + jnp.sum(cnt, axis=1, keepdims=True)
    rank_ref[...] = rank
    k = kc_ref[...]
    sbits = jnp.where(k < 0, jnp.bitwise_xor(k, jnp.int32(MININT)), jnp.bitwise_not(k))
    tanh_ref[...] = jnp.tanh(lax.bitcast_convert_type(sbits, jnp.float32))


def _rank(cand_key, cand_idx):
    return pl.pallas_call(
        _rank_body,
        out_shape=(jax.ShapeDtypeStruct((CMAX, 1), jnp.int32),
                   jax.ShapeDtypeStruct((CMAX, 1), jnp.float32)),
    )(cand_key.reshape(CMAX, 1), cand_idx.reshape(CMAX, 1),
      cand_key.reshape(1, CMAX), cand_idx.reshape(1, CMAX))


PSC = CMAX // 16


def _permute_gather_body(idx2d, rank2d, tanh2d, embs_hbm, rows_out, tanh_out,
                         civ, crv, ctv, tgt_v, idx_v, tanh_v, rows_v, zed_v,
                         zedf_v, sem, sh_sidx, sh_stanh):
    cid = lax.axis_index("c")
    sid = lax.axis_index("s")
    lanes = jnp.arange(16, dtype=jnp.int32)

    for c in range(8):
        zed_v[pl.ds(c * 16, 16)] = jnp.zeros((16,), jnp.int32)
        zedf_v[pl.ds(c * 16, 16)] = jnp.zeros((16,), jnp.float32)
    zbase = sid * 128
    pltpu.sync_copy(zed_v, sh_sidx.at[pl.ds(zbase, 128)])
    pltpu.sync_copy(zedf_v, sh_stanh.at[pl.ds(zbase, 128)])
    plsc.subcore_barrier()

    r0 = sid * 2
    pltpu.sync_copy(idx2d.at[pl.ds(r0, 2)], civ)
    pltpu.sync_copy(rank2d.at[pl.ds(r0, 2)], crv)
    pltpu.sync_copy(tanh2d.at[pl.ds(r0, 2)], ctv)

    for r in range(2):
        for c in range(5):
            rk = crv[r, pl.ds(c * 16, 16)]
            gpos = (r0 + r) * 80 + c * 16 + lanes
            ok = jnp.logical_and(rk >= 0, rk < K)
            tgt_v[r, pl.ds(c * 16, 16)] = jnp.where(ok, rk, K + gpos)
    for r in range(2):
        pltpu.sync_copy(civ.at[r], sh_sidx.at[tgt_v.at[r]])
        pltpu.sync_copy(ctv.at[r], sh_stanh.at[tgt_v.at[r]])
    plsc.subcore_barrier()

    base = cid * 1024 + sid * 64
    pltpu.sync_copy(sh_sidx.at[pl.ds(base, 64)], idx_v)
    pltpu.sync_copy(sh_stanh.at[pl.ds(base, 64)], tanh_v)
    for c in range(4):
        iv = idx_v[pl.ds(c * 16, 16)]
        idx_v[pl.ds(c * 16, 16)] = jnp.clip(iv, 0, N - 1)
    pltpu.async_copy(embs_hbm.at[idx_v], rows_v, sem).wait()
    pltpu.sync_copy(rows_v, rows_out.at[pl.ds(base, 64)])
    pltpu.sync_copy(tanh_v, tanh_out.at[pl.ds(base, 64)])


def _permute_gather(cand_idx, ranks, tanhs, node_embs):
    mesh = plsc.VectorSubcoreMesh(core_axis_name="c", subcore_axis_name="s")
    fn = functools.partial(
        pl.kernel,
        mesh=mesh,
        compiler_params=pltpu.CompilerParams(needs_layout_passes=False),
        out_type=(jax.ShapeDtypeStruct((K, D), jnp.float32),
                  jax.ShapeDtypeStruct((K,), jnp.float32)),
        scratch_types=[
            pltpu.VMEM((2, 80), jnp.int32),
            pltpu.VMEM((2, 80), jnp.int32),
            pltpu.VMEM((2, 80), jnp.float32),
            pltpu.VMEM((2, 80), jnp.int32),
            pltpu.VMEM((64,), jnp.int32),
            pltpu.VMEM((64,), jnp.float32),
            pltpu.VMEM((64, D), jnp.float32),
            pltpu.VMEM((128,), jnp.int32),
            pltpu.VMEM((128,), jnp.float32),
            pltpu.SemaphoreType.DMA,
            pltpu.VMEM_SHARED((K + CMAX,), jnp.int32),
            pltpu.VMEM_SHARED((K + CMAX,), jnp.float32),
        ],
    )(_permute_gather_body)
    return fn(cand_idx.reshape(CMAX // 80, 80), ranks.reshape(CMAX // 80, 80),
              tanhs.reshape(CMAX // 80, 80), node_embs)


def _scale_t_body(rows_ref, tanh_ref, out_ref):
    out_ref[...] = jnp.transpose(rows_ref[...] * tanh_ref[...], (1, 0))


def _scale_transpose(rows, sorted_tanh):
    return pl.pallas_call(
        _scale_t_body,
        out_shape=jax.ShapeDtypeStruct((D, K), jnp.float32),
    )(rows, sorted_tanh.reshape(K, 1))


def kernel(node_embs, scorer):
    norm = jnp.maximum(jnp.linalg.norm(scorer), 1e-6)
    scores = _scores(node_embs, scorer, norm).reshape(-1)
    scores_p = jnp.concatenate(
        [scores, jnp.full((NP - N,), -jnp.inf, jnp.float32)])
    cand_key, cand_idx = _select(scores_p)
    ranks, tanhs = _rank(cand_key, cand_idx)
    rows, sorted_tanh = _permute_gather(cand_idx, ranks.reshape(-1),
                                        tanhs.reshape(-1), node_embs)
    return _scale_transpose(rows, sorted_tanh)
